# baseline (speedup 1.0000x reference)
import functools

import jax
import jax.numpy as jnp
from jax import lax
from jax.experimental import pallas as pl
from jax.experimental.pallas import tpu as pltpu
from jax.experimental.pallas import tpu_sc as plsc

NUM_CORES = 2
NUM_SUBCORES = 16
NUM_WORKERS = NUM_CORES * NUM_SUBCORES
LANES = 16

ROWS_PER_GROUP = 4
CHUNK = 4096


def _build_sc_call(batch, input_size, total_syn):
    assert batch % (NUM_WORKERS * ROWS_PER_GROUP) == 0
    assert total_syn % CHUNK == 0 and CHUNK % LANES == 0
    rows_per_worker = batch // NUM_WORKERS
    groups = rows_per_worker // ROWS_PER_GROUP
    chunks = total_syn // CHUNK

    mesh = plsc.VectorSubcoreMesh(
        core_axis_name="c", subcore_axis_name="s", num_cores=NUM_CORES
    )

    @functools.partial(
        pl.kernel,
        out_type=jax.ShapeDtypeStruct((batch, total_syn), jnp.float32),
        mesh=mesh,
        compiler_params=pltpu.CompilerParams(needs_layout_passes=False),
        scratch_types=[
            *[pltpu.VMEM((input_size,), jnp.float32) for _ in range(ROWS_PER_GROUP)],
            pltpu.VMEM((CHUNK,), jnp.int32),
            pltpu.VMEM((CHUNK,), jnp.int32),
            *[pltpu.VMEM((CHUNK,), jnp.float32) for _ in range(ROWS_PER_GROUP)],
        ],
    )
    def sc_call(x_hbm, ip_hbm, iq_hbm, out_hbm, r0, r1, r2, r3, ipv, iqv,
                o0, o1, o2, o3):
        rows = [r0, r1, r2, r3]
        outs = [o0, o1, o2, o3]
        wid = lax.axis_index("s") * NUM_CORES + lax.axis_index("c")
        base = wid * rows_per_worker

        def group_body(g, carry):
            row0 = base + g * ROWS_PER_GROUP
            for r in range(ROWS_PER_GROUP):
                pltpu.sync_copy(x_hbm.at[row0 + r], rows[r])

            def chunk_body(c, carry):
                off = c * CHUNK
                pltpu.sync_copy(ip_hbm.at[pl.ds(off, CHUNK)], ipv)
                pltpu.sync_copy(iq_hbm.at[pl.ds(off, CHUNK)], iqv)

                def vec_body(i, carry):
                    ip = ipv[pl.ds(i * LANES, LANES)]
                    iq = iqv[pl.ds(i * LANES, LANES)]
                    for r in range(ROWS_PER_GROUP):
                        u = plsc.load_gather(rows[r], [ip])
                        v = plsc.load_gather(rows[r], [iq])
                        outs[r][pl.ds(i * LANES, LANES)] = jnp.where(
                            u > v, jnp.float32(1.0), jnp.float32(0.0)
                        )
                    return carry

                lax.fori_loop(0, CHUNK // LANES, vec_body, 0, unroll=8)
                for r in range(ROWS_PER_GROUP):
                    pltpu.sync_copy(
                        outs[r], out_hbm.at[row0 + r, pl.ds(off, CHUNK)]
                    )
                return carry

            return lax.fori_loop(0, chunks, chunk_body, carry, unroll=False)

        lax.fori_loop(0, groups, group_body, 0, unroll=False)

    return sc_call


def kernel(x, idx_p, idx_q):
    batch, input_size = x.shape
    num_neurons, num_synapses = idx_p.shape
    total_syn = num_neurons * num_synapses
    ip = idx_p.reshape(total_syn).astype(jnp.int32)
    iq = idx_q.reshape(total_syn).astype(jnp.int32)
    sc_call = _build_sc_call(batch, input_size, total_syn)
    out = sc_call(x, ip, iq)
    return out.reshape(batch, num_neurons, num_synapses)


# speedup vs baseline: 1.6914x; 1.6914x over previous
import functools

import jax
import jax.numpy as jnp
from jax import lax
from jax.experimental import pallas as pl
from jax.experimental.pallas import tpu as pltpu
from jax.experimental.pallas import tpu_sc as plsc

NUM_CORES = 2
NUM_SUBCORES = 16
NUM_WORKERS = NUM_CORES * NUM_SUBCORES
LANES = 16

ROWS_PER_GROUP = 4
CHUNK = 4096


def _build_sc_call(batch, input_size, total_syn):
    assert batch % (NUM_WORKERS * ROWS_PER_GROUP) == 0
    assert total_syn % CHUNK == 0 and CHUNK % LANES == 0
    rows_per_worker = batch // NUM_WORKERS
    groups = rows_per_worker // ROWS_PER_GROUP
    chunks = total_syn // CHUNK

    mesh = plsc.VectorSubcoreMesh(
        core_axis_name="c", subcore_axis_name="s", num_cores=NUM_CORES
    )

    @functools.partial(
        pl.kernel,
        out_type=jax.ShapeDtypeStruct((batch, total_syn), jnp.float32),
        mesh=mesh,
        compiler_params=pltpu.CompilerParams(needs_layout_passes=False),
        scratch_types=[
            *[pltpu.VMEM((input_size,), jnp.float32) for _ in range(ROWS_PER_GROUP)],
            pltpu.VMEM((CHUNK,), jnp.int32),
            pltpu.VMEM((CHUNK,), jnp.int32),
            *[pltpu.VMEM((CHUNK,), jnp.float32) for _ in range(ROWS_PER_GROUP)],
        ],
    )
    def sc_call(x_hbm, ip_hbm, iq_hbm, out_hbm, r0, r1, r2, r3, ipv, iqv,
                o0, o1, o2, o3):
        rows = [r0, r1, r2, r3]
        outs = [o0, o1, o2, o3]
        wid = lax.axis_index("s") * NUM_CORES + lax.axis_index("c")
        base = wid * rows_per_worker

        def group_body(g, carry):
            row0 = base + g * ROWS_PER_GROUP
            for r in range(ROWS_PER_GROUP):
                pltpu.sync_copy(x_hbm.at[row0 + r], rows[r])

            def chunk_body(c, carry):
                off = c * CHUNK
                pltpu.sync_copy(ip_hbm.at[pl.ds(off, CHUNK)], ipv)
                pltpu.sync_copy(iq_hbm.at[pl.ds(off, CHUNK)], iqv)

                @plsc.parallel_loop(0, CHUNK, LANES, unroll=2)
                def vec_body(i):
                    ip = ipv[pl.ds(i, LANES)]
                    iq = iqv[pl.ds(i, LANES)]
                    for r in range(ROWS_PER_GROUP):
                        u = plsc.load_gather(rows[r], [ip])
                        v = plsc.load_gather(rows[r], [iq])
                        outs[r][pl.ds(i, LANES)] = jnp.where(
                            u > v, jnp.float32(1.0), jnp.float32(0.0)
                        )
                for r in range(ROWS_PER_GROUP):
                    pltpu.sync_copy(
                        outs[r], out_hbm.at[row0 + r, pl.ds(off, CHUNK)]
                    )
                return carry

            return lax.fori_loop(0, chunks, chunk_body, carry, unroll=False)

        lax.fori_loop(0, groups, group_body, 0, unroll=False)

    return sc_call


def kernel(x, idx_p, idx_q):
    batch, input_size = x.shape
    num_neurons, num_synapses = idx_p.shape
    total_syn = num_neurons * num_synapses
    ip = idx_p.reshape(total_syn).astype(jnp.int32)
    iq = idx_q.reshape(total_syn).astype(jnp.int32)
    sc_call = _build_sc_call(batch, input_size, total_syn)
    out = sc_call(x, ip, iq)
    return out.reshape(batch, num_neurons, num_synapses)


# speedup vs baseline: 1.6969x; 1.0032x over previous
import functools

import jax
import jax.numpy as jnp
from jax import lax
from jax.experimental import pallas as pl
from jax.experimental.pallas import tpu as pltpu
from jax.experimental.pallas import tpu_sc as plsc

NUM_CORES = 2
NUM_SUBCORES = 16
NUM_WORKERS = NUM_CORES * NUM_SUBCORES
LANES = 16

ROWS_PER_GROUP = 4
CHUNK = 4096


def _build_sc_call(batch, input_size, total_syn):
    assert batch % (NUM_WORKERS * ROWS_PER_GROUP) == 0
    assert total_syn % CHUNK == 0 and CHUNK % LANES == 0
    rows_per_worker = batch // NUM_WORKERS
    groups = rows_per_worker // ROWS_PER_GROUP
    chunks = total_syn // CHUNK

    mesh = plsc.VectorSubcoreMesh(
        core_axis_name="c", subcore_axis_name="s", num_cores=NUM_CORES
    )

    @functools.partial(
        pl.kernel,
        out_type=jax.ShapeDtypeStruct((batch, total_syn), jnp.float32),
        mesh=mesh,
        compiler_params=pltpu.CompilerParams(needs_layout_passes=False),
        scratch_types=[
            *[pltpu.VMEM((input_size,), jnp.float32) for _ in range(ROWS_PER_GROUP)],
            pltpu.VMEM((CHUNK,), jnp.int32),
            pltpu.VMEM((CHUNK,), jnp.int32),
            *[pltpu.VMEM((CHUNK,), jnp.float32) for _ in range(ROWS_PER_GROUP)],
        ],
    )
    def sc_call(x_hbm, ip_hbm, iq_hbm, out_hbm, r0, r1, r2, r3, ipv, iqv,
                o0, o1, o2, o3):
        rows = [r0, r1, r2, r3]
        outs = [o0, o1, o2, o3]
        wid = lax.axis_index("s") * NUM_CORES + lax.axis_index("c")
        base = wid * rows_per_worker

        def group_body(g, carry):
            row0 = base + g * ROWS_PER_GROUP
            for r in range(ROWS_PER_GROUP):
                pltpu.sync_copy(x_hbm.at[row0 + r], rows[r])

            def chunk_body(c, carry):
                off = c * CHUNK
                pltpu.sync_copy(ip_hbm.at[pl.ds(off, CHUNK)], ipv)
                pltpu.sync_copy(iq_hbm.at[pl.ds(off, CHUNK)], iqv)

                @plsc.parallel_loop(0, CHUNK, LANES, unroll=4)
                def vec_body(i):
                    ip = ipv[pl.ds(i, LANES)]
                    iq = iqv[pl.ds(i, LANES)]
                    for r in range(ROWS_PER_GROUP):
                        u = plsc.load_gather(rows[r], [ip])
                        v = plsc.load_gather(rows[r], [iq])
                        outs[r][pl.ds(i, LANES)] = jnp.where(
                            u > v, jnp.float32(1.0), jnp.float32(0.0)
                        )
                for r in range(ROWS_PER_GROUP):
                    pltpu.sync_copy(
                        outs[r], out_hbm.at[row0 + r, pl.ds(off, CHUNK)]
                    )
                return carry

            return lax.fori_loop(0, chunks, chunk_body, carry, unroll=False)

        lax.fori_loop(0, groups, group_body, 0, unroll=False)

    return sc_call


def kernel(x, idx_p, idx_q):
    batch, input_size = x.shape
    num_neurons, num_synapses = idx_p.shape
    total_syn = num_neurons * num_synapses
    ip = idx_p.reshape(total_syn).astype(jnp.int32)
    iq = idx_q.reshape(total_syn).astype(jnp.int32)
    sc_call = _build_sc_call(batch, input_size, total_syn)
    out = sc_call(x, ip, iq)
    return out.reshape(batch, num_neurons, num_synapses)


# speedup vs baseline: 2.4637x; 1.4519x over previous
import functools

import jax
import jax.numpy as jnp
from jax import lax
from jax.experimental import pallas as pl
from jax.experimental.pallas import tpu as pltpu
from jax.experimental.pallas import tpu_sc as plsc

NUM_CORES = 2
NUM_SUBCORES = 16
NUM_WORKERS = NUM_CORES * NUM_SUBCORES
LANES = 16

ROWS_PER_GROUP = 4
CHUNK = 4096


def _build_sc_call(batch, input_size, total_syn):
    assert batch % (NUM_WORKERS * ROWS_PER_GROUP) == 0
    assert total_syn % (2 * CHUNK) == 0 and CHUNK % LANES == 0
    rows_per_worker = batch // NUM_WORKERS
    groups = rows_per_worker // ROWS_PER_GROUP
    chunks = total_syn // CHUNK

    mesh = plsc.VectorSubcoreMesh(
        core_axis_name="c", subcore_axis_name="s", num_cores=NUM_CORES
    )

    @functools.partial(
        pl.kernel,
        out_type=jax.ShapeDtypeStruct((batch, total_syn), jnp.float32),
        mesh=mesh,
        compiler_params=pltpu.CompilerParams(needs_layout_passes=False),
        scratch_types=[
            *[pltpu.VMEM((input_size,), jnp.float32) for _ in range(ROWS_PER_GROUP)],
            *[pltpu.VMEM((CHUNK,), jnp.int32) for _ in range(4)],
            *[pltpu.VMEM((CHUNK,), jnp.float32)
              for _ in range(2 * ROWS_PER_GROUP)],
            pltpu.SemaphoreType.DMA,
            *[pltpu.SemaphoreType.DMA for _ in range(2)],
            *[pltpu.SemaphoreType.DMA for _ in range(2)],
        ],
    )
    def sc_call(x_hbm, ip_hbm, iq_hbm, out_hbm, r0, r1, r2, r3,
                ip0, iq0, ip1, iq1, oa0, oa1, oa2, oa3, ob0, ob1, ob2, ob3,
                row_sem, is0, is1, os0, os1):
        rows = [r0, r1, r2, r3]
        idx_bufs = [(ip0, iq0), (ip1, iq1)]
        idx_sems = [is0, is1]
        out_bufs = [[oa0, oa1, oa2, oa3], [ob0, ob1, ob2, ob3]]
        out_sems = [os0, os1]
        wid = lax.axis_index("s") * NUM_CORES + lax.axis_index("c")
        base = wid * rows_per_worker

        def start_idx(c, buf):
            off = c * CHUNK
            pltpu.async_copy(ip_hbm.at[pl.ds(off, CHUNK)], idx_bufs[buf][0],
                             idx_sems[buf])
            pltpu.async_copy(iq_hbm.at[pl.ds(off, CHUNK)], idx_bufs[buf][1],
                             idx_sems[buf])

        def wait_idx(c, buf):
            off = c * CHUNK
            pltpu.make_async_copy(ip_hbm.at[pl.ds(off, CHUNK)],
                                  idx_bufs[buf][0], idx_sems[buf]).wait()
            pltpu.make_async_copy(iq_hbm.at[pl.ds(off, CHUNK)],
                                  idx_bufs[buf][1], idx_sems[buf]).wait()

        def start_out(row0, c, buf):
            for r in range(ROWS_PER_GROUP):
                pltpu.async_copy(out_bufs[buf][r],
                                 out_hbm.at[row0 + r, pl.ds(c * CHUNK, CHUNK)],
                                 out_sems[buf])

        def wait_out(row0, c, buf):
            for r in range(ROWS_PER_GROUP):
                pltpu.make_async_copy(
                    out_bufs[buf][r],
                    out_hbm.at[row0 + r, pl.ds(c * CHUNK, CHUNK)],
                    out_sems[buf]).wait()

        def compute_chunk(buf):
            ipv, iqv = idx_bufs[buf]
            obs = out_bufs[buf]

            @plsc.parallel_loop(0, CHUNK, LANES, unroll=4)
            def vec_body(i):
                ip = ipv[pl.ds(i, LANES)]
                iq = iqv[pl.ds(i, LANES)]
                for r in range(ROWS_PER_GROUP):
                    u = plsc.load_gather(rows[r], [ip])
                    v = plsc.load_gather(rows[r], [iq])
                    obs[r][pl.ds(i, LANES)] = jnp.where(
                        u > v, jnp.float32(1.0), jnp.float32(0.0)
                    )

        def group_body(g, carry):
            row0 = base + g * ROWS_PER_GROUP
            for r in range(ROWS_PER_GROUP):
                pltpu.async_copy(x_hbm.at[row0 + r], rows[r], row_sem)
            for r in range(ROWS_PER_GROUP):
                pltpu.make_async_copy(x_hbm.at[row0 + r], rows[r],
                                      row_sem).wait()
            start_idx(0, 0)
            start_idx(1, 1)

            def pair_body(c2, carry):
                c = c2 * 2
                for buf in range(2):
                    wait_idx(c + buf, buf)

                    @pl.when(c2 > 0)
                    def _():
                        wait_out(row0, c + buf - 2, buf)

                    compute_chunk(buf)
                    start_out(row0, c + buf, buf)

                    @pl.when(c2 < chunks // 2 - 1)
                    def _():
                        start_idx(c + buf + 2, buf)

                return carry

            lax.fori_loop(0, chunks // 2, pair_body, carry, unroll=False)
            for buf in range(2):
                wait_out(row0, chunks - 2 + buf, buf)
            return carry

        lax.fori_loop(0, groups, group_body, 0, unroll=False)

    return sc_call


def kernel(x, idx_p, idx_q):
    batch, input_size = x.shape
    num_neurons, num_synapses = idx_p.shape
    total_syn = num_neurons * num_synapses
    ip = idx_p.reshape(total_syn).astype(jnp.int32)
    iq = idx_q.reshape(total_syn).astype(jnp.int32)
    sc_call = _build_sc_call(batch, input_size, total_syn)
    out = sc_call(x, ip, iq)
    return out.reshape(batch, num_neurons, num_synapses)
